# baseline (speedup 1.0000x reference)
import functools

import jax
import jax.numpy as jnp
from jax import lax
from jax.experimental import pallas as pl
from jax.experimental.pallas import tpu as pltpu
from jax.experimental.pallas import tpu_sc as plsc

B = 16384
D = 64
PAIR = 2 * D
NPAIR = 500000
NC = 2
NS = 16
NW = NC * NS
B_PER_W = B // NW
CHUNK = 128
NCHUNK = B_PER_W // CHUNK


def _sc_gather_body(uidx_hbm, iidx_hbm, tu_mlp, ti_mlp, tu_mf, ti_mf,
                    ou_mlp, oi_mlp, ou_mf, oi_mf,
                    uidx_v, iidx_v, bu_mlp, bi_mlp, bu_mf, bi_mf,
                    s0, s1, s2, s3):
    wid = lax.axis_index("s") * NC + lax.axis_index("c")
    base = wid * B_PER_W
    for c in range(NCHUNK):
        off = base + c * CHUNK
        pltpu.sync_copy(uidx_hbm.at[pl.ds(off, CHUNK)], uidx_v)
        pltpu.sync_copy(iidx_hbm.at[pl.ds(off, CHUNK)], iidx_v)
        for j in range(CHUNK // 16):
            uidx_v[pl.ds(j * 16, 16)] = uidx_v[pl.ds(j * 16, 16)] >> 1
            iidx_v[pl.ds(j * 16, 16)] = iidx_v[pl.ds(j * 16, 16)] >> 1
        d0 = pltpu.async_copy(tu_mlp.at[uidx_v], bu_mlp, s0)
        d1 = pltpu.async_copy(ti_mlp.at[iidx_v], bi_mlp, s1)
        d2 = pltpu.async_copy(tu_mf.at[uidx_v], bu_mf, s2)
        d3 = pltpu.async_copy(ti_mf.at[iidx_v], bi_mf, s3)
        d0.wait()
        pltpu.sync_copy(bu_mlp, ou_mlp.at[pl.ds(off, CHUNK)])
        d1.wait()
        pltpu.sync_copy(bi_mlp, oi_mlp.at[pl.ds(off, CHUNK)])
        d2.wait()
        pltpu.sync_copy(bu_mf, ou_mf.at[pl.ds(off, CHUNK)])
        d3.wait()
        pltpu.sync_copy(bi_mf, oi_mf.at[pl.ds(off, CHUNK)])


_sc_gather = functools.partial(
    pl.kernel,
    out_type=[jax.ShapeDtypeStruct((B, PAIR), jnp.float32)] * 4,
    mesh=plsc.VectorSubcoreMesh(core_axis_name="c", subcore_axis_name="s",
                                num_cores=NC, num_subcores=NS),
    scratch_types=[
        pltpu.VMEM((CHUNK,), jnp.int32),
        pltpu.VMEM((CHUNK,), jnp.int32),
        pltpu.VMEM((CHUNK, PAIR), jnp.float32),
        pltpu.VMEM((CHUNK, PAIR), jnp.float32),
        pltpu.VMEM((CHUNK, PAIR), jnp.float32),
        pltpu.VMEM((CHUNK, PAIR), jnp.float32),
        pltpu.SemaphoreType.DMA,
        pltpu.SemaphoreType.DMA,
        pltpu.SemaphoreType.DMA,
        pltpu.SemaphoreType.DMA,
    ],
    compiler_params=pltpu.CompilerParams(use_tc_tiling_on_sc=False),
)(_sc_gather_body)


def _mlp_body(uidx, iidx, pu_mlp, pi_mlp, pu_mf, pi_mf,
              w1a, w1b, b1, w2, b2, wa1, wa2, ba, out_ref):
    uodd = (uidx[...].T & 1) == 1
    iodd = (iidx[...].T & 1) == 1
    pu_mlp_v, pi_mlp_v = pu_mlp[...], pi_mlp[...]
    pu_mf_v, pi_mf_v = pu_mf[...], pi_mf[...]
    u_mlp = jnp.where(uodd, pu_mlp_v[:, D:], pu_mlp_v[:, :D])
    i_mlp = jnp.where(iodd, pi_mlp_v[:, D:], pi_mlp_v[:, :D])
    u_mf = jnp.where(uodd, pu_mf_v[:, D:], pu_mf_v[:, :D])
    i_mf = jnp.where(iodd, pi_mf_v[:, D:], pi_mf_v[:, :D])
    x = jnp.dot(u_mlp, w1a[...], preferred_element_type=jnp.float32)
    x = x + jnp.dot(i_mlp, w1b[...], preferred_element_type=jnp.float32)
    x = jnp.maximum(x + b1[...], 0.0)
    x = jnp.dot(x, w2[...], preferred_element_type=jnp.float32) + b2[...]
    x = jnp.maximum(x, 0.0)
    mf = u_mf * i_mf
    z = jnp.dot(x, wa1[...], preferred_element_type=jnp.float32)
    z = z + jnp.dot(mf, wa2[...], preferred_element_type=jnp.float32)
    z = z + ba[0, 0]
    out_ref[...] = (1.0 / (1.0 + jnp.exp(-z)))[:, 0]


def _run_mlp(uidx, iidx, p_u_mlp, p_i_mlp, p_u_mf, p_i_mf,
             W1, b1, W2, b2, Wa, ba):
    R = 2048
    grid = (B // R,)
    pair_spec = pl.BlockSpec((R, PAIR), lambda i: (i, 0))
    idx_spec = pl.BlockSpec((1, R), lambda i: (0, i))
    full = lambda shape: pl.BlockSpec(shape, lambda i: (0,) * len(shape))
    return pl.pallas_call(
        _mlp_body,
        grid=grid,
        in_specs=[idx_spec, idx_spec,
                  pair_spec, pair_spec, pair_spec, pair_spec,
                  full((64, 64)), full((64, 64)), full((1, 64)),
                  full((64, 32)), full((1, 32)),
                  full((32, 1)), full((64, 1)), full((1, 1))],
        out_specs=pl.BlockSpec((R,), lambda i: (i,)),
        out_shape=jax.ShapeDtypeStruct((B,), jnp.float32),
    )(uidx.reshape(1, B), iidx.reshape(1, B),
      p_u_mlp, p_i_mlp, p_u_mf, p_i_mf,
      W1[:64], W1[64:], b1.reshape(1, 64),
      W2, b2.reshape(1, 32),
      Wa[:32], Wa[32:], ba.reshape(1, 1))


def kernel(user_indices, item_indices, emb_user_mlp, emb_item_mlp,
           emb_user_mf, emb_item_mf, W1, b1, W2, b2, Wa, ba):
    p_u_mlp, p_i_mlp, p_u_mf, p_i_mf = _sc_gather(
        user_indices, item_indices,
        emb_user_mlp.reshape(NPAIR, PAIR), emb_item_mlp.reshape(NPAIR, PAIR),
        emb_user_mf.reshape(NPAIR, PAIR), emb_item_mf.reshape(NPAIR, PAIR))
    return _run_mlp(user_indices, item_indices,
                    p_u_mlp, p_i_mlp, p_u_mf, p_i_mf,
                    W1, b1, W2, b2, Wa, ba)


# speedup vs baseline: 1.0005x; 1.0005x over previous
import functools

import jax
import jax.numpy as jnp
from jax import lax
from jax.experimental import pallas as pl
from jax.experimental.pallas import tpu as pltpu
from jax.experimental.pallas import tpu_sc as plsc

B = 16384
D = 64
PAIR = 2 * D
NPAIR = 500000
NC = 2
NS = 16
NW = NC * NS
B_PER_W = B // NW
CHUNK = 128
NCHUNK = B_PER_W // CHUNK


def _sc_gather_body(uidx_hbm, iidx_hbm, tu_mlp, ti_mlp, tu_mf, ti_mf,
                    ou_mlp, oi_mlp, ou_mf, oi_mf,
                    uidx_v, iidx_v, bu_mlp, bi_mlp, bu_mf, bi_mf,
                    s0, s1, s2, s3):
    wid = lax.axis_index("s") * NC + lax.axis_index("c")
    base = wid * B_PER_W
    for c in range(NCHUNK):
        off = base + c * CHUNK
        pltpu.sync_copy(uidx_hbm.at[pl.ds(off, CHUNK)], uidx_v)
        pltpu.sync_copy(iidx_hbm.at[pl.ds(off, CHUNK)], iidx_v)
        d0 = pltpu.async_copy(tu_mlp.at[uidx_v], bu_mlp, s0)
        d1 = pltpu.async_copy(ti_mlp.at[iidx_v], bi_mlp, s1)
        d2 = pltpu.async_copy(tu_mf.at[uidx_v], bu_mf, s2)
        d3 = pltpu.async_copy(ti_mf.at[iidx_v], bi_mf, s3)
        d0.wait()
        pltpu.sync_copy(bu_mlp, ou_mlp.at[pl.ds(off, CHUNK)])
        d1.wait()
        pltpu.sync_copy(bi_mlp, oi_mlp.at[pl.ds(off, CHUNK)])
        d2.wait()
        pltpu.sync_copy(bu_mf, ou_mf.at[pl.ds(off, CHUNK)])
        d3.wait()
        pltpu.sync_copy(bi_mf, oi_mf.at[pl.ds(off, CHUNK)])


_sc_gather = functools.partial(
    pl.kernel,
    out_type=[jax.ShapeDtypeStruct((B, PAIR), jnp.float32)] * 4,
    mesh=plsc.VectorSubcoreMesh(core_axis_name="c", subcore_axis_name="s",
                                num_cores=NC, num_subcores=NS),
    scratch_types=[
        pltpu.VMEM((CHUNK,), jnp.int32),
        pltpu.VMEM((CHUNK,), jnp.int32),
        pltpu.VMEM((CHUNK, PAIR), jnp.float32),
        pltpu.VMEM((CHUNK, PAIR), jnp.float32),
        pltpu.VMEM((CHUNK, PAIR), jnp.float32),
        pltpu.VMEM((CHUNK, PAIR), jnp.float32),
        pltpu.SemaphoreType.DMA,
        pltpu.SemaphoreType.DMA,
        pltpu.SemaphoreType.DMA,
        pltpu.SemaphoreType.DMA,
    ],
    compiler_params=pltpu.CompilerParams(use_tc_tiling_on_sc=True),
)(_sc_gather_body)


def _mlp_body(uidx, iidx, pu_mlp, pi_mlp, pu_mf, pi_mf,
              w1a, w1b, b1, w2, b2, wa1, wa2, ba, out_ref):
    uodd = (uidx[...].T & 1) == 1
    iodd = (iidx[...].T & 1) == 1
    pu_mlp_v, pi_mlp_v = pu_mlp[...], pi_mlp[...]
    pu_mf_v, pi_mf_v = pu_mf[...], pi_mf[...]
    u_mlp = jnp.where(uodd, pu_mlp_v[:, D:], pu_mlp_v[:, :D])
    i_mlp = jnp.where(iodd, pi_mlp_v[:, D:], pi_mlp_v[:, :D])
    u_mf = jnp.where(uodd, pu_mf_v[:, D:], pu_mf_v[:, :D])
    i_mf = jnp.where(iodd, pi_mf_v[:, D:], pi_mf_v[:, :D])
    x = jnp.dot(u_mlp, w1a[...], preferred_element_type=jnp.float32)
    x = x + jnp.dot(i_mlp, w1b[...], preferred_element_type=jnp.float32)
    x = jnp.maximum(x + b1[...], 0.0)
    x = jnp.dot(x, w2[...], preferred_element_type=jnp.float32) + b2[...]
    x = jnp.maximum(x, 0.0)
    mf = u_mf * i_mf
    z = jnp.dot(x, wa1[...], preferred_element_type=jnp.float32)
    z = z + jnp.dot(mf, wa2[...], preferred_element_type=jnp.float32)
    z = z + ba[0, 0]
    out_ref[...] = (1.0 / (1.0 + jnp.exp(-z)))[:, 0]


def _run_mlp(uidx, iidx, p_u_mlp, p_i_mlp, p_u_mf, p_i_mf,
             W1, b1, W2, b2, Wa, ba):
    R = 2048
    grid = (B // R,)
    pair_spec = pl.BlockSpec((R, PAIR), lambda i: (i, 0))
    idx_spec = pl.BlockSpec((1, R), lambda i: (0, i))
    full = lambda shape: pl.BlockSpec(shape, lambda i: (0,) * len(shape))
    return pl.pallas_call(
        _mlp_body,
        grid=grid,
        in_specs=[idx_spec, idx_spec,
                  pair_spec, pair_spec, pair_spec, pair_spec,
                  full((64, 64)), full((64, 64)), full((1, 64)),
                  full((64, 32)), full((1, 32)),
                  full((32, 1)), full((64, 1)), full((1, 1))],
        out_specs=pl.BlockSpec((R,), lambda i: (i,)),
        out_shape=jax.ShapeDtypeStruct((B,), jnp.float32),
    )(uidx.reshape(1, B), iidx.reshape(1, B),
      p_u_mlp, p_i_mlp, p_u_mf, p_i_mf,
      W1[:64], W1[64:], b1.reshape(1, 64),
      W2, b2.reshape(1, 32),
      Wa[:32], Wa[32:], ba.reshape(1, 1))


def kernel(user_indices, item_indices, emb_user_mlp, emb_item_mlp,
           emb_user_mf, emb_item_mf, W1, b1, W2, b2, Wa, ba):
    pu_idx = jax.lax.shift_right_logical(user_indices, 1)
    pi_idx = jax.lax.shift_right_logical(item_indices, 1)
    p_u_mlp, p_i_mlp, p_u_mf, p_i_mf = _sc_gather(
        pu_idx, pi_idx,
        emb_user_mlp.reshape(NPAIR, PAIR), emb_item_mlp.reshape(NPAIR, PAIR),
        emb_user_mf.reshape(NPAIR, PAIR), emb_item_mf.reshape(NPAIR, PAIR))
    return _run_mlp(user_indices, item_indices,
                    p_u_mlp, p_i_mlp, p_u_mf, p_i_mf,
                    W1, b1, W2, b2, Wa, ba)


# speedup vs baseline: 3.3668x; 3.3652x over previous
import functools

import jax
import jax.numpy as jnp
from jax import lax
from jax.experimental import pallas as pl
from jax.experimental.pallas import tpu as pltpu
from jax.experimental.pallas import tpu_sc as plsc

B = 16384
D = 64
NROWS = 1000000
NC = 2
NS = 16
NW = NC * NS
B_PER_W = B // NW
GRP = 16
NGRP = B_PER_W // GRP
WIN = 640
PADDED = ((NROWS + 127) // 128) * 128
BASE_LIM = PADDED - WIN


def _sc_stream_body(su_hbm, si_hbm, pu_hbm, pi_hbm,
                    tu_mlp, ti_mlp, tu_mf, ti_mf,
                    ou_mlp, oi_mlp, ou_mf, oi_mf,
                    idx_v, perm_v, slab, sbuf, kbuf, sem, sem_out):
    wid = lax.axis_index("s") * NC + lax.axis_index("c")
    base = wid * B_PER_W

    def run_table(tbl, sorted_hbm, perm_hbm, out):
        pltpu.sync_copy(sorted_hbm.at[pl.ds(base, B_PER_W)], idx_v)
        pltpu.sync_copy(perm_hbm.at[pl.ds(base, B_PER_W)], perm_v)
        lane = lax.iota(jnp.int32, 16)

        def group_body(g, slab_base):
            gvec = idx_v[pl.ds(g * GRP, GRP)]
            for m in range(GRP):
                r = jnp.sum(jnp.where(lane == m, gvec, 0))
                need = r >= slab_base + WIN
                new_base = jnp.minimum((r >> 7) << 7, BASE_LIM)
                tgt = jnp.where(need, new_base, slab_base)

                @pl.when(need)
                def _fetch():
                    off = pl.multiple_of(new_base, 128)
                    pltpu.sync_copy(tbl.at[:, pl.ds(off, WIN)], slab)

                col = r - tgt
                cvec = jnp.full((16,), col, jnp.int32)
                for gg in range(4):
                    rows = lane + (16 * gg)
                    vals = plsc.load_gather(slab, [rows, cvec])
                    sbuf[m, pl.ds(16 * gg, 16)] = vals
                slab_base = tgt
            kbuf[...] = perm_v[pl.ds(g * GRP, GRP)]
            pltpu.async_copy(sbuf, out.at[kbuf], sem_out).wait()
            return slab_base

        lax.fori_loop(0, NGRP, group_body, jnp.int32(-(1 << 24)))

    run_table(tu_mlp, su_hbm, pu_hbm, ou_mlp)
    run_table(tu_mf, su_hbm, pu_hbm, ou_mf)
    run_table(ti_mlp, si_hbm, pi_hbm, oi_mlp)
    run_table(ti_mf, si_hbm, pi_hbm, oi_mf)


_sc_stream = functools.partial(
    pl.kernel,
    out_type=[jax.ShapeDtypeStruct((B, 128), jnp.float32)] * 4,
    mesh=plsc.VectorSubcoreMesh(core_axis_name="c", subcore_axis_name="s",
                                num_cores=NC, num_subcores=NS),
    scratch_types=[
        pltpu.VMEM((B_PER_W,), jnp.int32),
        pltpu.VMEM((B_PER_W,), jnp.int32),
        pltpu.VMEM((64, WIN), jnp.float32),
        pltpu.VMEM((GRP, 128), jnp.float32),
        pltpu.VMEM((GRP,), jnp.int32),
        pltpu.SemaphoreType.DMA,
        pltpu.SemaphoreType.DMA,
    ],
    compiler_params=pltpu.CompilerParams(use_tc_tiling_on_sc=True,
                                         needs_layout_passes=False),
)(_sc_stream_body)


def _mlp_body(pu_mlp, pi_mlp, pu_mf, pi_mf,
              w1a, w1b, b1, w2, b2, wa1, wa2, ba, out_ref):
    u_mlp = pu_mlp[:, :D]
    i_mlp = pi_mlp[:, :D]
    u_mf = pu_mf[:, :D]
    i_mf = pi_mf[:, :D]
    x = jnp.dot(u_mlp, w1a[...], preferred_element_type=jnp.float32)
    x = x + jnp.dot(i_mlp, w1b[...], preferred_element_type=jnp.float32)
    x = jnp.maximum(x + b1[...], 0.0)
    x = jnp.dot(x, w2[...], preferred_element_type=jnp.float32) + b2[...]
    x = jnp.maximum(x, 0.0)
    mf = u_mf * i_mf
    z = jnp.dot(x, wa1[...], preferred_element_type=jnp.float32)
    z = z + jnp.dot(mf, wa2[...], preferred_element_type=jnp.float32)
    z = z + ba[0, 0]
    out_ref[...] = (1.0 / (1.0 + jnp.exp(-z)))[:, 0]


def _run_mlp(p_u_mlp, p_i_mlp, p_u_mf, p_i_mf, W1, b1, W2, b2, Wa, ba):
    R = 2048
    grid = (B // R,)
    pair_spec = pl.BlockSpec((R, 128), lambda i: (i, 0))
    full = lambda shape: pl.BlockSpec(shape, lambda i: (0,) * len(shape))
    return pl.pallas_call(
        _mlp_body,
        grid=grid,
        in_specs=[pair_spec, pair_spec, pair_spec, pair_spec,
                  full((64, 64)), full((64, 64)), full((1, 64)),
                  full((64, 32)), full((1, 32)),
                  full((32, 1)), full((64, 1)), full((1, 1))],
        out_specs=pl.BlockSpec((R,), lambda i: (i,)),
        out_shape=jax.ShapeDtypeStruct((B,), jnp.float32),
    )(p_u_mlp, p_i_mlp, p_u_mf, p_i_mf,
      W1[:64], W1[64:], b1.reshape(1, 64),
      W2, b2.reshape(1, 32),
      Wa[:32], Wa[32:], ba.reshape(1, 1))


def kernel(user_indices, item_indices, emb_user_mlp, emb_item_mlp,
           emb_user_mf, emb_item_mf, W1, b1, W2, b2, Wa, ba):
    iota = lax.iota(jnp.int32, B)
    su, pu = lax.sort((user_indices, iota), num_keys=1)
    si, pi = lax.sort((item_indices, iota), num_keys=1)
    p_u_mlp, p_i_mlp, p_u_mf, p_i_mf = _sc_stream(
        su, si, pu, pi,
        emb_user_mlp.T, emb_item_mlp.T, emb_user_mf.T, emb_item_mf.T)
    return _run_mlp(p_u_mlp, p_i_mlp, p_u_mf, p_i_mf,
                    W1, b1, W2, b2, Wa, ba)


# speedup vs baseline: 3.7118x; 1.1025x over previous
import functools

import jax
import jax.numpy as jnp
from jax import lax
from jax.experimental import pallas as pl
from jax.experimental.pallas import tpu as pltpu
from jax.experimental.pallas import tpu_sc as plsc

B = 16384
D = 64
NROWS = 1000000
NC = 2
NS = 16
NW = NC * NS
B_PER_W = B // NW
GRP = 16
NGRP = B_PER_W // GRP
WIN = 640
PADDED = ((NROWS + 127) // 128) * 128
BASE_LIM = PADDED - WIN


def _sc_stream_body(su_hbm, si_hbm, pu_hbm, pi_hbm,
                    tu_mlp, ti_mlp, tu_mf, ti_mf,
                    ou_mlp, oi_mlp, ou_mf, oi_mf,
                    idx_v, perm_v, slab_a, slab_b, sbuf_a, sbuf_b, kbuf,
                    sem_a, sem_b, sem_out):
    wid = lax.axis_index("s") * NC + lax.axis_index("c")
    base = wid * B_PER_W

    def run_pair(tbl_a, tbl_b, sorted_hbm, perm_hbm, out_a, out_b):
        pltpu.sync_copy(sorted_hbm.at[pl.ds(base, B_PER_W)], idx_v)
        pltpu.sync_copy(perm_hbm.at[pl.ds(base, B_PER_W)], perm_v)
        lane = lax.iota(jnp.int32, 16)

        def group_body(g, slab_base):
            gvec = idx_v[pl.ds(g * GRP, GRP)]
            for m in range(GRP):
                r = jnp.sum(jnp.where(lane == m, gvec, 0))
                need = r >= slab_base + WIN
                new_base = jnp.minimum((r >> 7) << 7, BASE_LIM)
                tgt = jnp.where(need, new_base, slab_base)

                @pl.when(need)
                def _fetch():
                    off = pl.multiple_of(new_base, 128)
                    da = pltpu.async_copy(
                        tbl_a.at[:, pl.ds(off, WIN)], slab_a, sem_a)
                    db = pltpu.async_copy(
                        tbl_b.at[:, pl.ds(off, WIN)], slab_b, sem_b)
                    da.wait()
                    db.wait()

                col = r - tgt
                cvec = jnp.full((16,), col, jnp.int32)
                for gg in range(4):
                    rows = lane + (16 * gg)
                    va = plsc.load_gather(slab_a, [rows, cvec])
                    sbuf_a[m, pl.ds(16 * gg, 16)] = va
                    vb = plsc.load_gather(slab_b, [rows, cvec])
                    sbuf_b[m, pl.ds(16 * gg, 16)] = vb
                slab_base = tgt
            kbuf[...] = perm_v[pl.ds(g * GRP, GRP)]
            da = pltpu.async_copy(sbuf_a, out_a.at[kbuf], sem_out)
            db = pltpu.async_copy(sbuf_b, out_b.at[kbuf], sem_out)
            da.wait()
            db.wait()
            return slab_base

        lax.fori_loop(0, NGRP, group_body, jnp.int32(-(1 << 24)))

    run_pair(tu_mlp, tu_mf, su_hbm, pu_hbm, ou_mlp, ou_mf)
    run_pair(ti_mlp, ti_mf, si_hbm, pi_hbm, oi_mlp, oi_mf)


_sc_stream = functools.partial(
    pl.kernel,
    out_type=[jax.ShapeDtypeStruct((B, 128), jnp.float32)] * 4,
    mesh=plsc.VectorSubcoreMesh(core_axis_name="c", subcore_axis_name="s",
                                num_cores=NC, num_subcores=NS),
    scratch_types=[
        pltpu.VMEM((B_PER_W,), jnp.int32),
        pltpu.VMEM((B_PER_W,), jnp.int32),
        pltpu.VMEM((64, WIN), jnp.float32),
        pltpu.VMEM((64, WIN), jnp.float32),
        pltpu.VMEM((GRP, 128), jnp.float32),
        pltpu.VMEM((GRP, 128), jnp.float32),
        pltpu.VMEM((GRP,), jnp.int32),
        pltpu.SemaphoreType.DMA,
        pltpu.SemaphoreType.DMA,
        pltpu.SemaphoreType.DMA,
    ],
    compiler_params=pltpu.CompilerParams(use_tc_tiling_on_sc=True,
                                         needs_layout_passes=False),
)(_sc_stream_body)


def _mlp_body(pu_mlp, pi_mlp, pu_mf, pi_mf,
              w1a, w1b, b1, w2, b2, wa1, wa2, ba, out_ref):
    u_mlp = pu_mlp[:, :D]
    i_mlp = pi_mlp[:, :D]
    u_mf = pu_mf[:, :D]
    i_mf = pi_mf[:, :D]
    x = jnp.dot(u_mlp, w1a[...], preferred_element_type=jnp.float32)
    x = x + jnp.dot(i_mlp, w1b[...], preferred_element_type=jnp.float32)
    x = jnp.maximum(x + b1[...], 0.0)
    x = jnp.dot(x, w2[...], preferred_element_type=jnp.float32) + b2[...]
    x = jnp.maximum(x, 0.0)
    mf = u_mf * i_mf
    z = jnp.dot(x, wa1[...], preferred_element_type=jnp.float32)
    z = z + jnp.dot(mf, wa2[...], preferred_element_type=jnp.float32)
    z = z + ba[0, 0]
    out_ref[...] = (1.0 / (1.0 + jnp.exp(-z)))[:, 0]


def _run_mlp(p_u_mlp, p_i_mlp, p_u_mf, p_i_mf, W1, b1, W2, b2, Wa, ba):
    R = 2048
    grid = (B // R,)
    pair_spec = pl.BlockSpec((R, 128), lambda i: (i, 0))
    full = lambda shape: pl.BlockSpec(shape, lambda i: (0,) * len(shape))
    return pl.pallas_call(
        _mlp_body,
        grid=grid,
        in_specs=[pair_spec, pair_spec, pair_spec, pair_spec,
                  full((64, 64)), full((64, 64)), full((1, 64)),
                  full((64, 32)), full((1, 32)),
                  full((32, 1)), full((64, 1)), full((1, 1))],
        out_specs=pl.BlockSpec((R,), lambda i: (i,)),
        out_shape=jax.ShapeDtypeStruct((B,), jnp.float32),
    )(p_u_mlp, p_i_mlp, p_u_mf, p_i_mf,
      W1[:64], W1[64:], b1.reshape(1, 64),
      W2, b2.reshape(1, 32),
      Wa[:32], Wa[32:], ba.reshape(1, 1))


def kernel(user_indices, item_indices, emb_user_mlp, emb_item_mlp,
           emb_user_mf, emb_item_mf, W1, b1, W2, b2, Wa, ba):
    iota = lax.iota(jnp.int32, B)
    su, pu = lax.sort((user_indices, iota), num_keys=1)
    si, pi = lax.sort((item_indices, iota), num_keys=1)
    p_u_mlp, p_i_mlp, p_u_mf, p_i_mf = _sc_stream(
        su, si, pu, pi,
        emb_user_mlp.T, emb_item_mlp.T, emb_user_mf.T, emb_item_mf.T)
    return _run_mlp(p_u_mlp, p_i_mlp, p_u_mf, p_i_mf,
                    W1, b1, W2, b2, Wa, ba)
